# baseline (speedup 1.0000x reference)
import functools

import jax
import jax.numpy as jnp
from jax import lax
from jax.experimental import pallas as pl
from jax.experimental.pallas import tpu as pltpu
from jax.experimental.pallas import tpu_sc as plsc

BATCH = 4096
SEQ = 200
D = 64
HID = 128
CLS = 128

NC = 2
NS = 16
NW = NC * NS
LANES = 16

C = 2
G_SPLIT = ((0, 104), (104, 96))
ROWS_PER_W = BATCH // NW
CHUNKS_PER_W = ROWS_PER_W // C
NCOL = D // LANES


NBUF = 8
OSTAGE = 32


def _pool_body(x_hbm, emb, out, idx_all, rows0, rows1, rows2, rows3,
               rows4, rows5, rows6, rows7, out_stage,
               sem0, sem1, sem2, sem3, sem4, sem5, sem6, sem7):
    wid = lax.axis_index("s") * NC + lax.axis_index("c")
    base_row = wid * ROWS_PER_W
    bufs = (rows0, rows1, rows2, rows3, rows4, rows5, rows6, rows7)
    sems = (sem0, sem1, sem2, sem3, sem4, sem5, sem6, sem7)

    pltpu.sync_copy(x_hbm.at[pl.ds(base_row, ROWS_PER_W)], idx_all)

    def fire(g, buf, sem):
        for off, size in G_SPLIT:
            pltpu.async_copy(
                emb.at[idx_all.at[g, pl.ds(off, size)]],
                buf.at[pl.ds(off, size)], sem)

    def wait_all(buf, sem):
        pltpu.make_async_copy(emb.at[pl.ds(0, SEQ)], buf, sem).wait()

    def accumulate(g, buf):
        def red_body(r8, accs):
            accs = list(accs)
            for rr in range(8):
                r = r8 * 8 + rr
                s = (rr & 1) * NCOL
                for c in range(NCOL):
                    accs[s + c] = accs[s + c] + buf[
                        r, pl.ds(c * LANES, LANES)]
            return tuple(accs)

        accs = lax.fori_loop(
            0, SEQ // 8, red_body,
            tuple(jnp.zeros((LANES,), jnp.float32)
                  for _ in range(2 * NCOL)))
        gs = lax.rem(g, OSTAGE)
        for c in range(NCOL):
            out_stage[gs, pl.ds(c * LANES, LANES)] = (
                (accs[c] + accs[NCOL + c]) * (1.0 / SEQ))

    for k in range(NBUF):
        fire(k, bufs[k], sems[k])

    def outer(h, carry):
        for k in range(NBUF):
            g = NBUF * h + k
            wait_all(bufs[k], sems[k])
            accumulate(g, bufs[k])

            @pl.when(g + NBUF < ROWS_PER_W)
            def _():
                fire(g + NBUF, bufs[k], sems[k])

            @pl.when(lax.rem(g, OSTAGE) == OSTAGE - 1)
            def _():
                pltpu.sync_copy(
                    out_stage,
                    out.at[pl.ds(base_row + (g - (OSTAGE - 1)), OSTAGE)])
        return carry

    lax.fori_loop(0, ROWS_PER_W // NBUF, outer, 0)


@jax.jit
def _pool(x, emb):
    mesh = plsc.VectorSubcoreMesh(core_axis_name="c", subcore_axis_name="s")
    return pl.kernel(
        _pool_body,
        out_type=jax.ShapeDtypeStruct((BATCH, D), jnp.float32),
        mesh=mesh,
        compiler_params=pltpu.CompilerParams(use_tc_tiling_on_sc=False),
        scratch_types=(
            [pltpu.VMEM((ROWS_PER_W, SEQ), jnp.int32)]
            + [pltpu.VMEM((SEQ, D), jnp.float32) for _ in range(NBUF)]
            + [pltpu.VMEM((OSTAGE, D), jnp.float32)]
            + [pltpu.SemaphoreType.DMA for _ in range(NBUF)]
        ),
    )(x, emb)


def _mlp_body(p_ref, w1t_ref, b1_ref, w2t_ref, b2_ref, o_ref):
    h = jnp.dot(p_ref[...], w1t_ref[...],
                preferred_element_type=jnp.float32) + b1_ref[...]
    o_ref[...] = jnp.dot(h, w2t_ref[...],
                         preferred_element_type=jnp.float32) + b2_ref[...]


@jax.jit
def _mlp(pooled, W1t, b1, W2t, b2):
    return pl.pallas_call(
        _mlp_body,
        out_shape=jax.ShapeDtypeStruct((BATCH, CLS), jnp.float32),
    )(pooled, W1t, b1, W2t, b2)


def kernel(x, emb, W1, b1, W2, b2):
    pooled = _pool(x, emb)
    return _mlp(pooled, W1.T, b1[None, :], W2.T, b2[None, :])


# speedup vs baseline: 1.0064x; 1.0064x over previous
import functools

import jax
import jax.numpy as jnp
from jax import lax
from jax.experimental import pallas as pl
from jax.experimental.pallas import tpu as pltpu
from jax.experimental.pallas import tpu_sc as plsc

BATCH = 4096
SEQ = 200
D = 64
HID = 128
CLS = 128

NC = 2
NS = 16
NW = NC * NS
LANES = 16

C = 2
G_SPLIT = ((0, 104), (104, 96))
ROWS_PER_W = BATCH // NW
CHUNKS_PER_W = ROWS_PER_W // C
NCOL = D // LANES


NBUF = 4


def _pool_body(x_hbm, emb, out, idx_all, rows0, rows1, rows2, rows3,
               out_stage, sem0, sem1, sem2, sem3):
    wid = lax.axis_index("s") * NC + lax.axis_index("c")
    base_row = wid * ROWS_PER_W
    bufs = (rows0, rows1, rows2, rows3)
    sems = (sem0, sem1, sem2, sem3)

    pltpu.sync_copy(x_hbm.at[pl.ds(base_row, ROWS_PER_W)], idx_all)

    def fire(g, buf, sem):
        for off, size in G_SPLIT:
            pltpu.async_copy(
                emb.at[idx_all.at[g, pl.ds(off, size)]],
                buf.at[pl.ds(off, size)], sem)

    def wait_all(buf, sem):
        pltpu.make_async_copy(emb.at[pl.ds(0, SEQ)], buf, sem).wait()

    def accumulate(g, buf):
        def red_body(r8, accs):
            accs = list(accs)
            for rr in range(8):
                r = r8 * 8 + rr
                s = (rr & 1) * NCOL
                for c in range(NCOL):
                    accs[s + c] = accs[s + c] + buf[
                        r, pl.ds(c * LANES, LANES)]
            return tuple(accs)

        accs = lax.fori_loop(
            0, SEQ // 8, red_body,
            tuple(jnp.zeros((LANES,), jnp.float32)
                  for _ in range(2 * NCOL)))
        for c in range(NCOL):
            out_stage[g, pl.ds(c * LANES, LANES)] = (
                (accs[c] + accs[NCOL + c]) * (1.0 / SEQ))

    for k in range(NBUF):
        fire(k, bufs[k], sems[k])

    def outer(h, carry):
        for k in range(NBUF):
            g = NBUF * h + k
            wait_all(bufs[k], sems[k])
            accumulate(g, bufs[k])

            @pl.when(g + NBUF < ROWS_PER_W)
            def _():
                fire(g + NBUF, bufs[k], sems[k])
        return carry

    lax.fori_loop(0, ROWS_PER_W // NBUF, outer, 0)
    pltpu.sync_copy(out_stage, out.at[pl.ds(base_row, ROWS_PER_W)])


@jax.jit
def _pool(x, emb):
    mesh = plsc.VectorSubcoreMesh(core_axis_name="c", subcore_axis_name="s")
    return pl.kernel(
        _pool_body,
        out_type=jax.ShapeDtypeStruct((BATCH, D), jnp.float32),
        mesh=mesh,
        compiler_params=pltpu.CompilerParams(use_tc_tiling_on_sc=False),
        scratch_types=(
            [pltpu.VMEM((ROWS_PER_W, SEQ), jnp.int32)]
            + [pltpu.VMEM((SEQ, D), jnp.float32) for _ in range(NBUF)]
            + [pltpu.VMEM((ROWS_PER_W, D), jnp.float32)]
            + [pltpu.SemaphoreType.DMA for _ in range(NBUF)]
        ),
    )(x, emb)


def _mlp_body(p_ref, w1t_ref, b1_ref, w2t_ref, b2_ref, o_ref):
    h = jnp.dot(p_ref[...], w1t_ref[...],
                preferred_element_type=jnp.float32) + b1_ref[...]
    o_ref[...] = jnp.dot(h, w2t_ref[...],
                         preferred_element_type=jnp.float32) + b2_ref[...]


@jax.jit
def _mlp(pooled, W1t, b1, W2t, b2):
    return pl.pallas_call(
        _mlp_body,
        out_shape=jax.ShapeDtypeStruct((BATCH, CLS), jnp.float32),
    )(pooled, W1t, b1, W2t, b2)


def kernel(x, emb, W1, b1, W2, b2):
    pooled = _pool(x, emb)
    return _mlp(pooled, W1.T, b1[None, :], W2.T, b2[None, :])
